# baseline (speedup 1.0000x reference)
import functools

import jax
import jax.numpy as jnp
from jax import lax
from jax.experimental import pallas as pl
from jax.experimental.pallas import tpu as pltpu
from jax.experimental.pallas import tpu_sc as plsc

_NUM_CORES = 2
_NUM_SUBCORES = 16
_NW = _NUM_CORES * _NUM_SUBCORES

_PACK_BLK = 2048


@functools.lru_cache(maxsize=None)
def _make_pack_kernel(batch: int, n_sites: int):
    c_const = (1 << n_sites) - 1

    def body(x_ref, o_ref):
        xb = x_ref[...]
        j = lax.broadcasted_iota(jnp.int32, (1, n_sites), 1)
        w = jnp.int32(1 << (n_sites - 1)) >> j
        t = jnp.sum(xb * w, axis=1)
        o_ref[...] = (c_const - t) >> 1

    return pl.pallas_call(
        body,
        grid=(batch // _PACK_BLK,),
        in_specs=[pl.BlockSpec((_PACK_BLK, n_sites), lambda i: (i, 0))],
        out_specs=pl.BlockSpec((_PACK_BLK,), lambda i: (i,)),
        out_shape=jax.ShapeDtypeStruct((batch,), jnp.int32),
    )


@functools.lru_cache(maxsize=None)
def _make_gather_kernel(batch: int):
    b_per_w = batch // _NW
    assert batch % (8 * _NW) == 0
    mesh = plsc.VectorSubcoreMesh(
        core_axis_name="c", subcore_axis_name="s")

    @functools.partial(
        pl.kernel,
        out_type=(
            jax.ShapeDtypeStruct((batch,), jnp.float32),
            jax.ShapeDtypeStruct((batch,), jnp.float32),
        ),
        mesh=mesh,
        compiler_params=pltpu.CompilerParams(needs_layout_passes=False),
        scratch_types=[
            pltpu.VMEM((b_per_w,), jnp.int32),
            pltpu.VMEM((b_per_w,), jnp.float32),
            pltpu.VMEM((b_per_w,), jnp.float32),
            pltpu.SemaphoreType.DMA,
        ],
    )
    def sc_kernel(idx_hbm, real_hbm, imag_hbm, out_r, out_i,
                  idxv, rv, iv, sem):
        wid = lax.axis_index("s") * _NUM_CORES + lax.axis_index("c")
        base = wid * b_per_w
        pltpu.sync_copy(idx_hbm.at[pl.ds(base, b_per_w)], idxv)
        pltpu.async_copy(real_hbm.at[idxv], rv, sem).wait()
        pltpu.async_copy(imag_hbm.at[idxv], iv, sem).wait()
        pltpu.sync_copy(rv, out_r.at[pl.ds(base, b_per_w)])
        pltpu.sync_copy(iv, out_i.at[pl.ds(base, b_per_w)])

    return sc_kernel


def kernel(x, real, imag):
    batch, n_sites = x.shape
    idx = _make_pack_kernel(batch, n_sites)(x)
    r, i = _make_gather_kernel(batch)(idx, real, imag)
    return lax.complex(r, i)
